# Initial kernel scaffold; baseline (speedup 1.0000x reference)
import jax
import jax.numpy as jnp
from jax.experimental import pallas as pl


def kernel(u_id, pos_i_id, adj_matrix, edges, entity_embedding, W1, b1, g1, beta1, W2, b2, g2, beta2, dis_user_emb, dis_item_emb):
    raise NotImplementedError("write your pallas kernel here")


# speedup vs baseline: 1.0457x; 1.0457x over previous
import functools

import jax
import jax.numpy as jnp
import numpy as np
from jax.experimental import pallas as pl
from jax.experimental.pallas import tpu as pltpu

NN = 50000
NE = 800000
DD = 64
BB = 4096
KK = 32
HI = 24999
BLK = 512


def _sampling_noise():
    key = jax.random.key(42)
    k1, k2, k3 = jax.random.split(key, 3)
    g1 = jax.random.gumbel(k1, (BB, KK), dtype=jnp.float32)
    g2 = jax.random.gumbel(k2, (BB, KK), dtype=jnp.float32)
    rnd = jax.random.randint(k3, (BB, KK), 0, HI + 1, dtype=jnp.int32)
    return g1, g2, rnd


def _stage1_body(u_ref, pos_ref, ie_ref, hop_ref, g_ref, nid_ref, hop_out_ref):
    p = jnp.sum((pos_ref[...][:, None, :] * ie_ref[...])
                * u_ref[...][:, None, :], axis=-1)
    m = jnp.max(p, axis=1, keepdims=True)
    e = jnp.exp(p - m)
    probs = e / jnp.sum(e, axis=1, keepdims=True)
    score = jnp.log(probs + 1e-12) + g_ref[...]
    vmax = jnp.max(score, axis=1, keepdims=True)
    col = jax.lax.broadcasted_iota(jnp.int32, (BLK, KK), 1)
    nid = jnp.min(jnp.where(score == vmax, col, KK), axis=1, keepdims=True)
    nid_ref[...] = nid
    hop_out_ref[...] = jnp.sum(
        jnp.where(col == nid, hop_ref[...], 0), axis=1, keepdims=True)


def _stage1(u_e, pos_e, i_e, one_hop0, gumb1):
    grid = BB // BLK
    return pl.pallas_call(
        _stage1_body,
        grid=(grid,),
        in_specs=[
            pl.BlockSpec((BLK, DD), lambda i: (i, 0)),
            pl.BlockSpec((BLK, DD), lambda i: (i, 0)),
            pl.BlockSpec((BLK, KK, DD), lambda i: (i, 0, 0)),
            pl.BlockSpec((BLK, KK), lambda i: (i, 0)),
            pl.BlockSpec((BLK, KK), lambda i: (i, 0)),
        ],
        out_specs=[
            pl.BlockSpec((BLK, 1), lambda i: (i, 0)),
            pl.BlockSpec((BLK, 1), lambda i: (i, 0)),
        ],
        out_shape=[
            jax.ShapeDtypeStruct((BB, 1), jnp.int32),
            jax.ShapeDtypeStruct((BB, 1), jnp.int32),
        ],
    )(u_e, pos_e, i_e, one_hop0, gumb1)


def _stage2a_body(u_ref, pos_ref, ie_ref, hop_ref, g_ref, rnd_ref,
                  neg_ref, clog_ref):
    p = jnp.sum((pos_ref[...][:, None, :] * ie_ref[...])
                * u_ref[...][:, None, :], axis=-1)
    m = jnp.max(p, axis=1, keepdims=True)
    e = jnp.exp(p - m)
    probs = e / jnp.sum(e, axis=1, keepdims=True)
    score = jnp.log(probs + 1e-12) + g_ref[...]
    col = jax.lax.broadcasted_iota(jnp.int32, (BLK, KK), 1)
    hop = hop_ref[...]
    rnd = rnd_ref[...]
    neg_cols = []
    clog_cols = []
    s = score
    for j in range(KK):
        vmax = jnp.max(s, axis=1, keepdims=True)
        sel = jnp.min(jnp.where(s == vmax, col, KK), axis=1, keepdims=True)
        cand = jnp.sum(jnp.where(col == sel, hop, 0), axis=1, keepdims=True)
        clog = jnp.sum(jnp.where(col == sel, probs, 0.0), axis=1, keepdims=True)
        r = rnd[:, j][:, None]
        cand = jnp.where((cand > HI) | (cand < 0), r, cand)
        neg_cols.append(cand)
        clog_cols.append(clog)
        s = jnp.where(col == sel, -jnp.inf, s)
    neg_ref[...] = jnp.concatenate(neg_cols, axis=1)
    clog_ref[...] = jnp.concatenate(clog_cols, axis=1)


def _stage2a(u_e, pos2_e, i_e2, one_hop2, gumb2, rnd):
    grid = BB // BLK
    return pl.pallas_call(
        _stage2a_body,
        grid=(grid,),
        in_specs=[
            pl.BlockSpec((BLK, DD), lambda i: (i, 0)),
            pl.BlockSpec((BLK, DD), lambda i: (i, 0)),
            pl.BlockSpec((BLK, KK, DD), lambda i: (i, 0, 0)),
            pl.BlockSpec((BLK, KK), lambda i: (i, 0)),
            pl.BlockSpec((BLK, KK), lambda i: (i, 0)),
            pl.BlockSpec((BLK, KK), lambda i: (i, 0)),
        ],
        out_specs=[
            pl.BlockSpec((BLK, KK), lambda i: (i, 0)),
            pl.BlockSpec((BLK, KK), lambda i: (i, 0)),
        ],
        out_shape=[
            jax.ShapeDtypeStruct((BB, KK), jnp.int32),
            jax.ShapeDtypeStruct((BB, KK), jnp.float32),
        ],
    )(u_e, pos2_e, i_e2, one_hop2, gumb2, rnd)


def _stage2b_body(du_ref, di_ref, neg_ref, clog_ref, gneg_ref, glog_ref):
    rank = jnp.sum(di_ref[...] * du_ref[...][:, None, :], axis=-1)
    vmax = jnp.max(rank, axis=1, keepdims=True)
    col = jax.lax.broadcasted_iota(jnp.int32, (BLK, KK), 1)
    idx = jnp.min(jnp.where(rank == vmax, col, KK), axis=1, keepdims=True)
    gneg_ref[...] = jnp.sum(jnp.where(col == idx, neg_ref[...], 0),
                            axis=1, keepdims=True)
    glog_ref[...] = jnp.sum(jnp.where(col == idx, clog_ref[...], 0.0),
                            axis=1, keepdims=True)


def _stage2b(dis_u, dis_i, neg, clog):
    grid = BB // BLK
    return pl.pallas_call(
        _stage2b_body,
        grid=(grid,),
        in_specs=[
            pl.BlockSpec((BLK, DD), lambda i: (i, 0)),
            pl.BlockSpec((BLK, KK, DD), lambda i: (i, 0, 0)),
            pl.BlockSpec((BLK, KK), lambda i: (i, 0)),
            pl.BlockSpec((BLK, KK), lambda i: (i, 0)),
        ],
        out_specs=[
            pl.BlockSpec((BLK, 1), lambda i: (i, 0)),
            pl.BlockSpec((BLK, 1), lambda i: (i, 0)),
        ],
        out_shape=[
            jax.ShapeDtypeStruct((BB, 1), jnp.int32),
            jax.ShapeDtypeStruct((BB, 1), jnp.float32),
        ],
    )(dis_u, dis_i, neg, clog)


def _conv(x, src, dst, W, b, n):
    xw = x @ W
    loop = jnp.arange(n)
    s = jnp.concatenate([src, loop])
    d = jnp.concatenate([dst, loop])
    deg = jax.ops.segment_sum(jnp.ones(d.shape, dtype=xw.dtype), d, num_segments=n)
    dinv = jnp.where(deg > 0, deg ** -0.5, 0.0)
    norm = dinv[s] * dinv[d]
    msg = xw[s] * norm[:, None]
    out = jax.ops.segment_sum(msg, d, num_segments=n)
    return out + b


def _bn(x, g, beta, eps=1e-5):
    m = jnp.mean(x, axis=0)
    v = jnp.var(x, axis=0)
    return (x - m) / jnp.sqrt(v + eps) * g + beta


def _gcn(x, edges, W1, b1, g1, beta1, W2, b2, g2, beta2):
    src, dst = edges[0], edges[1]
    h = _conv(x, src, dst, W1, b1, NN)
    h = jax.nn.leaky_relu(h, 0.01)
    h = _bn(h, g1, beta1)
    h = _conv(h, src, dst, W2, b2, NN)
    h = _bn(h, g2, beta2)
    return h


def kernel(u_id, pos_i_id, adj_matrix, edges, entity_embedding,
           W1, b1, g1, beta1, W2, b2, g2, beta2,
           dis_user_emb, dis_item_emb):
    gumb1, gumb2, rnd = _sampling_noise()
    emb = _gcn(entity_embedding, edges, W1, b1, g1, beta1, W2, b2, g2, beta2)
    u_e = jnp.take(emb, u_id, axis=0)
    pos_e = jnp.take(emb, pos_i_id, axis=0)
    one_hop0 = jnp.take(adj_matrix, pos_i_id, axis=0)
    i_e = jnp.take(emb, one_hop0.reshape(-1), axis=0).reshape(BB, KK, DD)
    _, one_hop = _stage1(u_e, pos_e, i_e, one_hop0, gumb1)
    one_hop = one_hop[:, 0]
    pos2_e = jnp.take(emb, one_hop, axis=0)
    one_hop2 = jnp.take(adj_matrix, one_hop, axis=0)
    i_e2 = jnp.take(emb, one_hop2.reshape(-1), axis=0).reshape(BB, KK, DD)
    neg, clog = _stage2a(u_e, pos2_e, i_e2, one_hop2, gumb2, rnd)
    dis_u = jnp.take(dis_user_emb, u_id, axis=0)
    dis_i = jnp.take(dis_item_emb, neg.reshape(-1), axis=0).reshape(BB, KK, DD)
    gneg, glog = _stage2b(dis_u, dis_i, neg, clog)
    return gneg[:, 0], glog[:, 0]


# speedup vs baseline: 1.2328x; 1.1790x over previous
import functools

import jax
import jax.numpy as jnp
import numpy as np
from jax import lax
from jax.experimental import pallas as pl
from jax.experimental.pallas import tpu as pltpu
from jax.experimental.pallas import tpu_sc as plsc

NN = 50000
NE = 800000
DD = 64
BB = 4096
KK = 32
HI = 24999
BLK = 512


def _sampling_noise():
    key = jax.random.key(42)
    k1, k2, k3 = jax.random.split(key, 3)
    g1 = jax.random.gumbel(k1, (BB, KK), dtype=jnp.float32)
    g2 = jax.random.gumbel(k2, (BB, KK), dtype=jnp.float32)
    rnd = jax.random.randint(k3, (BB, KK), 0, HI + 1, dtype=jnp.int32)
    return g1, g2, rnd


def _stage1_body(u_ref, pos_ref, ie_ref, hop_ref, g_ref, nid_ref, hop_out_ref):
    p = jnp.sum((pos_ref[...][:, None, :] * ie_ref[...])
                * u_ref[...][:, None, :], axis=-1)
    m = jnp.max(p, axis=1, keepdims=True)
    e = jnp.exp(p - m)
    probs = e / jnp.sum(e, axis=1, keepdims=True)
    score = jnp.log(probs + 1e-12) + g_ref[...]
    vmax = jnp.max(score, axis=1, keepdims=True)
    col = jax.lax.broadcasted_iota(jnp.int32, (BLK, KK), 1)
    nid = jnp.min(jnp.where(score == vmax, col, KK), axis=1, keepdims=True)
    nid_ref[...] = nid
    hop_out_ref[...] = jnp.sum(
        jnp.where(col == nid, hop_ref[...], 0), axis=1, keepdims=True)


def _stage1(u_e, pos_e, i_e, one_hop0, gumb1):
    grid = BB // BLK
    return pl.pallas_call(
        _stage1_body,
        grid=(grid,),
        in_specs=[
            pl.BlockSpec((BLK, DD), lambda i: (i, 0)),
            pl.BlockSpec((BLK, DD), lambda i: (i, 0)),
            pl.BlockSpec((BLK, KK, DD), lambda i: (i, 0, 0)),
            pl.BlockSpec((BLK, KK), lambda i: (i, 0)),
            pl.BlockSpec((BLK, KK), lambda i: (i, 0)),
        ],
        out_specs=[
            pl.BlockSpec((BLK, 1), lambda i: (i, 0)),
            pl.BlockSpec((BLK, 1), lambda i: (i, 0)),
        ],
        out_shape=[
            jax.ShapeDtypeStruct((BB, 1), jnp.int32),
            jax.ShapeDtypeStruct((BB, 1), jnp.int32),
        ],
    )(u_e, pos_e, i_e, one_hop0, gumb1)


def _stage2a_body(u_ref, pos_ref, ie_ref, hop_ref, g_ref, rnd_ref,
                  neg_ref, clog_ref):
    p = jnp.sum((pos_ref[...][:, None, :] * ie_ref[...])
                * u_ref[...][:, None, :], axis=-1)
    m = jnp.max(p, axis=1, keepdims=True)
    e = jnp.exp(p - m)
    probs = e / jnp.sum(e, axis=1, keepdims=True)
    score = jnp.log(probs + 1e-12) + g_ref[...]
    col = jax.lax.broadcasted_iota(jnp.int32, (BLK, KK), 1)
    hop = hop_ref[...]
    rnd = rnd_ref[...]
    neg_cols = []
    clog_cols = []
    s = score
    for j in range(KK):
        vmax = jnp.max(s, axis=1, keepdims=True)
        sel = jnp.min(jnp.where(s == vmax, col, KK), axis=1, keepdims=True)
        cand = jnp.sum(jnp.where(col == sel, hop, 0), axis=1, keepdims=True)
        clog = jnp.sum(jnp.where(col == sel, probs, 0.0), axis=1, keepdims=True)
        r = rnd[:, j][:, None]
        cand = jnp.where((cand > HI) | (cand < 0), r, cand)
        neg_cols.append(cand)
        clog_cols.append(clog)
        s = jnp.where(col == sel, -jnp.inf, s)
    neg_ref[...] = jnp.concatenate(neg_cols, axis=1)
    clog_ref[...] = jnp.concatenate(clog_cols, axis=1)


def _stage2a(u_e, pos2_e, i_e2, one_hop2, gumb2, rnd):
    grid = BB // BLK
    return pl.pallas_call(
        _stage2a_body,
        grid=(grid,),
        in_specs=[
            pl.BlockSpec((BLK, DD), lambda i: (i, 0)),
            pl.BlockSpec((BLK, DD), lambda i: (i, 0)),
            pl.BlockSpec((BLK, KK, DD), lambda i: (i, 0, 0)),
            pl.BlockSpec((BLK, KK), lambda i: (i, 0)),
            pl.BlockSpec((BLK, KK), lambda i: (i, 0)),
            pl.BlockSpec((BLK, KK), lambda i: (i, 0)),
        ],
        out_specs=[
            pl.BlockSpec((BLK, KK), lambda i: (i, 0)),
            pl.BlockSpec((BLK, KK), lambda i: (i, 0)),
        ],
        out_shape=[
            jax.ShapeDtypeStruct((BB, KK), jnp.int32),
            jax.ShapeDtypeStruct((BB, KK), jnp.float32),
        ],
    )(u_e, pos2_e, i_e2, one_hop2, gumb2, rnd)


def _stage2b_body(du_ref, di_ref, neg_ref, clog_ref, gneg_ref, glog_ref):
    rank = jnp.sum(di_ref[...] * du_ref[...][:, None, :], axis=-1)
    vmax = jnp.max(rank, axis=1, keepdims=True)
    col = jax.lax.broadcasted_iota(jnp.int32, (BLK, KK), 1)
    idx = jnp.min(jnp.where(rank == vmax, col, KK), axis=1, keepdims=True)
    gneg_ref[...] = jnp.sum(jnp.where(col == idx, neg_ref[...], 0),
                            axis=1, keepdims=True)
    glog_ref[...] = jnp.sum(jnp.where(col == idx, clog_ref[...], 0.0),
                            axis=1, keepdims=True)


def _stage2b(dis_u, dis_i, neg, clog):
    grid = BB // BLK
    return pl.pallas_call(
        _stage2b_body,
        grid=(grid,),
        in_specs=[
            pl.BlockSpec((BLK, DD), lambda i: (i, 0)),
            pl.BlockSpec((BLK, KK, DD), lambda i: (i, 0, 0)),
            pl.BlockSpec((BLK, KK), lambda i: (i, 0)),
            pl.BlockSpec((BLK, KK), lambda i: (i, 0)),
        ],
        out_specs=[
            pl.BlockSpec((BLK, 1), lambda i: (i, 0)),
            pl.BlockSpec((BLK, 1), lambda i: (i, 0)),
        ],
        out_shape=[
            jax.ShapeDtypeStruct((BB, 1), jnp.int32),
            jax.ShapeDtypeStruct((BB, 1), jnp.float32),
        ],
    )(dis_u, dis_i, neg, clog)


NW = 32


@functools.lru_cache(maxsize=None)
def _sc_mesh():
    return plsc.VectorSubcoreMesh(core_axis_name="c", subcore_axis_name="s")


def _sc_gather_rows(table, idx, blk):
    M = idx.shape[0]
    Dw = table.shape[1]
    per = M // NW
    nb = per // blk
    assert per * NW == M and nb * blk == per, (M, blk)

    @functools.partial(
        pl.kernel,
        mesh=_sc_mesh(),
        compiler_params=pltpu.CompilerParams(use_tc_tiling_on_sc=False),
        out_type=jax.ShapeDtypeStruct((M, Dw), table.dtype),
        scratch_types=[
            pltpu.VMEM((blk,), jnp.int32),
            pltpu.VMEM((blk, Dw), table.dtype),
            pltpu.SemaphoreType.DMA,
        ],
    )
    def k(table_hbm, idx_hbm, out_hbm, idx_v, rows_v, sem):
        wid = lax.axis_index("s") * 2 + lax.axis_index("c")
        base = wid * per

        def body(b, carry):
            off = base + b * blk
            pltpu.sync_copy(idx_hbm.at[pl.ds(off, blk)], idx_v)
            pltpu.async_copy(table_hbm.at[idx_v], rows_v, sem).wait()
            pltpu.sync_copy(rows_v, out_hbm.at[pl.ds(off, blk)])
            return carry

        lax.fori_loop(0, nb, body, 0)

    return k(table, idx)


def _gather64(table, idx):
    M = idx.shape[0]
    step = NW * 1024 if M >= NW * 1024 else NW * 8
    M_pad = ((M + step - 1) // step) * step
    if M_pad != M:
        idx = jnp.concatenate([idx, jnp.zeros((M_pad - M,), jnp.int32)])
    blk = 1024 if M_pad >= NW * 1024 else max(8, M_pad // NW)
    out = _sc_gather_rows(table, idx, blk)
    return out[:M] if M_pad != M else out


def _conv(x, src, dst, W, b, n):
    xw = x @ W
    loop = jnp.arange(n)
    s = jnp.concatenate([src, loop])
    d = jnp.concatenate([dst, loop])
    deg = jax.ops.segment_sum(jnp.ones(d.shape, dtype=xw.dtype), d, num_segments=n)
    dinv = jnp.where(deg > 0, deg ** -0.5, 0.0)
    norm = dinv[s] * dinv[d]
    msg = _gather64(xw, s) * norm[:, None]
    out = jax.ops.segment_sum(msg, d, num_segments=n)
    return out + b


def _bn(x, g, beta, eps=1e-5):
    m = jnp.mean(x, axis=0)
    v = jnp.var(x, axis=0)
    return (x - m) / jnp.sqrt(v + eps) * g + beta


def _gcn(x, edges, W1, b1, g1, beta1, W2, b2, g2, beta2):
    src, dst = edges[0], edges[1]
    h = _conv(x, src, dst, W1, b1, NN)
    h = jax.nn.leaky_relu(h, 0.01)
    h = _bn(h, g1, beta1)
    h = _conv(h, src, dst, W2, b2, NN)
    h = _bn(h, g2, beta2)
    return h


def kernel(u_id, pos_i_id, adj_matrix, edges, entity_embedding,
           W1, b1, g1, beta1, W2, b2, g2, beta2,
           dis_user_emb, dis_item_emb):
    gumb1, gumb2, rnd = _sampling_noise()
    emb = _gcn(entity_embedding, edges, W1, b1, g1, beta1, W2, b2, g2, beta2)
    u_e = _gather64(emb, u_id)
    pos_e = _gather64(emb, pos_i_id)
    one_hop0 = _sc_gather_rows(adj_matrix, pos_i_id, BB // NW)
    i_e = _gather64(emb, one_hop0.reshape(-1)).reshape(BB, KK, DD)
    _, one_hop = _stage1(u_e, pos_e, i_e, one_hop0, gumb1)
    one_hop = one_hop[:, 0]
    pos2_e = _gather64(emb, one_hop)
    one_hop2 = _sc_gather_rows(adj_matrix, one_hop, BB // NW)
    i_e2 = _gather64(emb, one_hop2.reshape(-1)).reshape(BB, KK, DD)
    neg, clog = _stage2a(u_e, pos2_e, i_e2, one_hop2, gumb2, rnd)
    dis_u = _gather64(dis_user_emb, u_id)
    dis_i = _gather64(dis_item_emb, neg.reshape(-1)).reshape(BB, KK, DD)
    gneg, glog = _stage2b(dis_u, dis_i, neg, clog)
    return gneg[:, 0], glog[:, 0]


# speedup vs baseline: 2.3606x; 1.9149x over previous
import functools

import jax
import jax.numpy as jnp
import numpy as np
from jax import lax
from jax.experimental import pallas as pl
from jax.experimental.pallas import tpu as pltpu
from jax.experimental.pallas import tpu_sc as plsc

NN = 50000
NE = 800000
DD = 64
BB = 4096
KK = 32
HI = 24999
BLK = 512


def _sampling_noise():
    key = jax.random.key(42)
    k1, k2, k3 = jax.random.split(key, 3)
    g1 = jax.random.gumbel(k1, (BB, KK), dtype=jnp.float32)
    g2 = jax.random.gumbel(k2, (BB, KK), dtype=jnp.float32)
    rnd = jax.random.randint(k3, (BB, KK), 0, HI + 1, dtype=jnp.int32)
    return g1, g2, rnd


def _stage1_body(u_ref, pos_ref, ie_ref, hop_ref, g_ref, nid_ref, hop_out_ref):
    p = jnp.sum((pos_ref[...][:, None, :] * ie_ref[...])
                * u_ref[...][:, None, :], axis=-1)
    m = jnp.max(p, axis=1, keepdims=True)
    e = jnp.exp(p - m)
    probs = e / jnp.sum(e, axis=1, keepdims=True)
    score = jnp.log(probs + 1e-12) + g_ref[...]
    vmax = jnp.max(score, axis=1, keepdims=True)
    col = jax.lax.broadcasted_iota(jnp.int32, (BLK, KK), 1)
    nid = jnp.min(jnp.where(score == vmax, col, KK), axis=1, keepdims=True)
    nid_ref[...] = nid
    hop_out_ref[...] = jnp.sum(
        jnp.where(col == nid, hop_ref[...], 0), axis=1, keepdims=True)


def _stage1(u_e, pos_e, i_e, one_hop0, gumb1):
    grid = BB // BLK
    return pl.pallas_call(
        _stage1_body,
        grid=(grid,),
        in_specs=[
            pl.BlockSpec((BLK, DD), lambda i: (i, 0)),
            pl.BlockSpec((BLK, DD), lambda i: (i, 0)),
            pl.BlockSpec((BLK, KK, DD), lambda i: (i, 0, 0)),
            pl.BlockSpec((BLK, KK), lambda i: (i, 0)),
            pl.BlockSpec((BLK, KK), lambda i: (i, 0)),
        ],
        out_specs=[
            pl.BlockSpec((BLK, 1), lambda i: (i, 0)),
            pl.BlockSpec((BLK, 1), lambda i: (i, 0)),
        ],
        out_shape=[
            jax.ShapeDtypeStruct((BB, 1), jnp.int32),
            jax.ShapeDtypeStruct((BB, 1), jnp.int32),
        ],
    )(u_e, pos_e, i_e, one_hop0, gumb1)


def _stage2a_body(u_ref, pos_ref, ie_ref, hop_ref, g_ref, rnd_ref,
                  neg_ref, clog_ref):
    p = jnp.sum((pos_ref[...][:, None, :] * ie_ref[...])
                * u_ref[...][:, None, :], axis=-1)
    m = jnp.max(p, axis=1, keepdims=True)
    e = jnp.exp(p - m)
    probs = e / jnp.sum(e, axis=1, keepdims=True)
    score = jnp.log(probs + 1e-12) + g_ref[...]
    col = jax.lax.broadcasted_iota(jnp.int32, (BLK, KK), 1)
    hop = hop_ref[...]
    rnd = rnd_ref[...]
    neg_cols = []
    clog_cols = []
    s = score
    for j in range(KK):
        vmax = jnp.max(s, axis=1, keepdims=True)
        sel = jnp.min(jnp.where(s == vmax, col, KK), axis=1, keepdims=True)
        cand = jnp.sum(jnp.where(col == sel, hop, 0), axis=1, keepdims=True)
        clog = jnp.sum(jnp.where(col == sel, probs, 0.0), axis=1, keepdims=True)
        r = rnd[:, j][:, None]
        cand = jnp.where((cand > HI) | (cand < 0), r, cand)
        neg_cols.append(cand)
        clog_cols.append(clog)
        s = jnp.where(col == sel, -jnp.inf, s)
    neg_ref[...] = jnp.concatenate(neg_cols, axis=1)
    clog_ref[...] = jnp.concatenate(clog_cols, axis=1)


def _stage2a(u_e, pos2_e, i_e2, one_hop2, gumb2, rnd):
    grid = BB // BLK
    return pl.pallas_call(
        _stage2a_body,
        grid=(grid,),
        in_specs=[
            pl.BlockSpec((BLK, DD), lambda i: (i, 0)),
            pl.BlockSpec((BLK, DD), lambda i: (i, 0)),
            pl.BlockSpec((BLK, KK, DD), lambda i: (i, 0, 0)),
            pl.BlockSpec((BLK, KK), lambda i: (i, 0)),
            pl.BlockSpec((BLK, KK), lambda i: (i, 0)),
            pl.BlockSpec((BLK, KK), lambda i: (i, 0)),
        ],
        out_specs=[
            pl.BlockSpec((BLK, KK), lambda i: (i, 0)),
            pl.BlockSpec((BLK, KK), lambda i: (i, 0)),
        ],
        out_shape=[
            jax.ShapeDtypeStruct((BB, KK), jnp.int32),
            jax.ShapeDtypeStruct((BB, KK), jnp.float32),
        ],
    )(u_e, pos2_e, i_e2, one_hop2, gumb2, rnd)


def _stage2b_body(du_ref, di_ref, neg_ref, clog_ref, gneg_ref, glog_ref):
    rank = jnp.sum(di_ref[...] * du_ref[...][:, None, :], axis=-1)
    vmax = jnp.max(rank, axis=1, keepdims=True)
    col = jax.lax.broadcasted_iota(jnp.int32, (BLK, KK), 1)
    idx = jnp.min(jnp.where(rank == vmax, col, KK), axis=1, keepdims=True)
    gneg_ref[...] = jnp.sum(jnp.where(col == idx, neg_ref[...], 0),
                            axis=1, keepdims=True)
    glog_ref[...] = jnp.sum(jnp.where(col == idx, clog_ref[...], 0.0),
                            axis=1, keepdims=True)


def _stage2b(dis_u, dis_i, neg, clog):
    grid = BB // BLK
    return pl.pallas_call(
        _stage2b_body,
        grid=(grid,),
        in_specs=[
            pl.BlockSpec((BLK, DD), lambda i: (i, 0)),
            pl.BlockSpec((BLK, KK, DD), lambda i: (i, 0, 0)),
            pl.BlockSpec((BLK, KK), lambda i: (i, 0)),
            pl.BlockSpec((BLK, KK), lambda i: (i, 0)),
        ],
        out_specs=[
            pl.BlockSpec((BLK, 1), lambda i: (i, 0)),
            pl.BlockSpec((BLK, 1), lambda i: (i, 0)),
        ],
        out_shape=[
            jax.ShapeDtypeStruct((BB, 1), jnp.int32),
            jax.ShapeDtypeStruct((BB, 1), jnp.float32),
        ],
    )(dis_u, dis_i, neg, clog)


NW = 32


@functools.lru_cache(maxsize=None)
def _sc_mesh():
    return plsc.VectorSubcoreMesh(core_axis_name="c", subcore_axis_name="s")


def _sc_gather_rows(table, idx, blk):
    M = idx.shape[0]
    Dw = table.shape[1]
    per = M // NW
    nb = per // blk
    assert per * NW == M and nb * blk == per, (M, blk)

    @functools.partial(
        pl.kernel,
        mesh=_sc_mesh(),
        compiler_params=pltpu.CompilerParams(use_tc_tiling_on_sc=False),
        out_type=jax.ShapeDtypeStruct((M, Dw), table.dtype),
        scratch_types=[
            pltpu.VMEM((blk,), jnp.int32),
            pltpu.VMEM((blk, Dw), table.dtype),
            pltpu.SemaphoreType.DMA,
        ],
    )
    def k(table_hbm, idx_hbm, out_hbm, idx_v, rows_v, sem):
        wid = lax.axis_index("s") * 2 + lax.axis_index("c")
        base = wid * per

        def body(b, carry):
            off = base + b * blk
            pltpu.sync_copy(idx_hbm.at[pl.ds(off, blk)], idx_v)
            pltpu.async_copy(table_hbm.at[idx_v], rows_v, sem).wait()
            pltpu.sync_copy(rows_v, out_hbm.at[pl.ds(off, blk)])
            return carry

        lax.fori_loop(0, nb, body, 0)

    return k(table, idx)


def _gather64(table, idx):
    M = idx.shape[0]
    step = NW * 1024 if M >= NW * 1024 else NW * 8
    M_pad = ((M + step - 1) // step) * step
    if M_pad != M:
        idx = jnp.concatenate([idx, jnp.zeros((M_pad - M,), jnp.int32)])
    blk = 1024 if M_pad >= NW * 1024 else max(8, M_pad // NW)
    out = _sc_gather_rows(table, idx, blk)
    return out[:M] if M_pad != M else out


NBIN = 50016
MPAD = NW * 26 * 1024
EBLK = 1024


def _dup_stats(d_pad):
    d2 = d_pad.reshape(-1, 16)
    eq = d2[:, :, None] == d2[:, None, :]
    lane = jnp.arange(16, dtype=jnp.int32)
    before = lane[None, :, None] > lane[None, None, :]
    duprank = jnp.sum(eq & before, axis=2, dtype=jnp.int32)
    after = lane[None, :, None] < lane[None, None, :]
    islast = 1 - jnp.max((eq & after).astype(jnp.int32), axis=2)
    return duprank.reshape(-1), islast.reshape(-1)


def _sc_hist(d_pad, duprank, islast):
    per = MPAD // NW
    nb = per // EBLK

    @functools.partial(
        pl.kernel,
        mesh=_sc_mesh(),
        compiler_params=pltpu.CompilerParams(use_tc_tiling_on_sc=False, needs_layout_passes=False),
        out_type=jax.ShapeDtypeStruct((NW, NBIN), jnp.int32),
        scratch_types=[
            pltpu.VMEM((EBLK,), jnp.int32),
            pltpu.VMEM((EBLK,), jnp.int32),
            pltpu.VMEM((EBLK,), jnp.int32),
            pltpu.VMEM((NBIN,), jnp.int32),
        ],
    )
    def k(d_hbm, dr_hbm, il_hbm, out_hbm, d_v, dr_v, il_v, hist_v):
        wid = lax.axis_index("s") * 2 + lax.axis_index("c")
        base = wid * per

        def zero_body(i, c):
            hist_v[pl.ds(i * 16, 16)] = jnp.zeros((16,), jnp.int32)
            return c

        lax.fori_loop(0, NBIN // 16, zero_body, 0)

        def blk_body(bI, c):
            off = base + bI * EBLK
            pltpu.sync_copy(d_hbm.at[pl.ds(off, EBLK)], d_v)
            pltpu.sync_copy(dr_hbm.at[pl.ds(off, EBLK)], dr_v)
            pltpu.sync_copy(il_hbm.at[pl.ds(off, EBLK)], il_v)

            def e_body(j, c2):
                d16 = d_v[pl.ds(j * 16, 16)]
                cnt = plsc.load_gather(hist_v, [d16])
                cnt = cnt + dr_v[pl.ds(j * 16, 16)] + 1
                mask = il_v[pl.ds(j * 16, 16)] != 0
                plsc.store_scatter(hist_v, [d16], cnt, mask=mask)
                return c2

            lax.fori_loop(0, EBLK // 16, e_body, 0)
            return c

        lax.fori_loop(0, nb, blk_body, 0)
        pltpu.sync_copy(hist_v, out_hbm.at[wid])

    return k(d_pad, duprank, islast)


def _sc_rank_permute(d_pad, s_pad, duprank, islast, dinv_p, basew):
    per = MPAD // NW
    nb = per // EBLK

    @functools.partial(
        pl.kernel,
        mesh=_sc_mesh(),
        compiler_params=pltpu.CompilerParams(use_tc_tiling_on_sc=False, needs_layout_passes=False),
        out_type=[
            jax.ShapeDtypeStruct((MPAD,), jnp.int32),
            jax.ShapeDtypeStruct((MPAD,), jnp.int32),
            jax.ShapeDtypeStruct((MPAD,), jnp.float32),
        ],
        scratch_types=[
            pltpu.VMEM((EBLK,), jnp.int32),
            pltpu.VMEM((EBLK,), jnp.int32),
            pltpu.VMEM((EBLK,), jnp.int32),
            pltpu.VMEM((EBLK,), jnp.int32),
            pltpu.VMEM((EBLK,), jnp.int32),
            pltpu.VMEM((EBLK,), jnp.float32),
            pltpu.VMEM((NBIN,), jnp.int32),
            pltpu.VMEM((NBIN,), jnp.float32),
            pltpu.SemaphoreType.DMA,
            pltpu.SemaphoreType.DMA,
            pltpu.SemaphoreType.DMA,
        ],
    )
    def k(d_hbm, s_hbm, dr_hbm, il_hbm, dinv_hbm, basew_hbm,
          ss_hbm, ds_hbm, ns_hbm,
          d_v, s_v, dr_v, il_v, pos_v, norm_v, base_v, dinv_v,
          sem1, sem2, sem3):
        wid = lax.axis_index("s") * 2 + lax.axis_index("c")
        base = wid * per
        pltpu.sync_copy(basew_hbm.at[wid], base_v)
        pltpu.sync_copy(dinv_hbm, dinv_v)

        def blk_body(bI, c):
            off = base + bI * EBLK
            pltpu.sync_copy(d_hbm.at[pl.ds(off, EBLK)], d_v)
            pltpu.sync_copy(s_hbm.at[pl.ds(off, EBLK)], s_v)
            pltpu.sync_copy(dr_hbm.at[pl.ds(off, EBLK)], dr_v)
            pltpu.sync_copy(il_hbm.at[pl.ds(off, EBLK)], il_v)

            def e_body(j, c2):
                sl = pl.ds(j * 16, 16)
                d16 = d_v[sl]
                s16 = s_v[sl]
                p16 = plsc.load_gather(base_v, [d16]) + dr_v[sl]
                mask = il_v[sl] != 0
                plsc.store_scatter(base_v, [d16], p16 + 1, mask=mask)
                pos_v[sl] = p16
                ds_ = plsc.load_gather(dinv_v, [s16])
                dd_ = plsc.load_gather(dinv_v, [d16])
                norm_v[sl] = ds_ * dd_
                return c2

            lax.fori_loop(0, EBLK // 16, e_body, 0)
            h1 = pltpu.async_copy(s_v, ss_hbm.at[pos_v], sem1)
            h2 = pltpu.async_copy(d_v, ds_hbm.at[pos_v], sem2)
            h3 = pltpu.async_copy(norm_v, ns_hbm.at[pos_v], sem3)
            h1.wait()
            h2.wait()
            h3.wait()
            return c

        lax.fori_loop(0, nb, blk_body, 0)

    return k(d_pad, s_pad, duprank, islast, dinv_p, basew)


def _sorted_edges(src, dst):
    loop = jnp.arange(NN)
    npad = MPAD - (NE + NN)
    d_pad = jnp.concatenate([dst, loop, jnp.full((npad,), NBIN - 2, jnp.int32)])
    s_pad = jnp.concatenate([src, loop, jnp.zeros((npad,), jnp.int32)])
    duprank, islast = _dup_stats(d_pad)
    hist = _sc_hist(d_pad, duprank, islast)
    tot = jnp.sum(hist, axis=0)
    deg = tot[:NN].astype(jnp.float32)
    dinv = jnp.where(deg > 0, deg ** -0.5, 0.0)
    dinv_p = jnp.concatenate([dinv, jnp.zeros((NBIN - NN,), jnp.float32)])
    off = jnp.cumsum(tot) - tot
    basew = off[None, :] + (jnp.cumsum(hist, axis=0) - hist)
    ss, ds, ns = _sc_rank_permute(d_pad, s_pad, duprank, islast, dinv_p, basew)
    return ss, ds[: NE + NN], ns[: NE + NN]


def _conv(x, W, b, s_sorted, d_sorted, norm_sorted):
    xw = x @ W
    msg = _gather64(xw, s_sorted)[: NE + NN] * norm_sorted[:, None]
    out = lax.scatter_add(
        jnp.zeros((NN, DD), jnp.float32),
        d_sorted[:, None],
        msg,
        lax.ScatterDimensionNumbers(
            update_window_dims=(1,),
            inserted_window_dims=(0,),
            scatter_dims_to_operand_dims=(0,),
        ),
        indices_are_sorted=True,
        unique_indices=False,
    )
    return out + b


def _bn(x, g, beta, eps=1e-5):
    m = jnp.mean(x, axis=0)
    v = jnp.var(x, axis=0)
    return (x - m) / jnp.sqrt(v + eps) * g + beta


def _gcn(x, edges, W1, b1, g1, beta1, W2, b2, g2, beta2):
    src, dst = edges[0], edges[1]
    ss, ds, ns = _sorted_edges(src, dst)
    h = _conv(x, W1, b1, ss, ds, ns)
    h = jax.nn.leaky_relu(h, 0.01)
    h = _bn(h, g1, beta1)
    h = _conv(h, W2, b2, ss, ds, ns)
    h = _bn(h, g2, beta2)
    return h


def kernel(u_id, pos_i_id, adj_matrix, edges, entity_embedding,
           W1, b1, g1, beta1, W2, b2, g2, beta2,
           dis_user_emb, dis_item_emb):
    gumb1, gumb2, rnd = _sampling_noise()
    emb = _gcn(entity_embedding, edges, W1, b1, g1, beta1, W2, b2, g2, beta2)
    u_e = _gather64(emb, u_id)
    pos_e = _gather64(emb, pos_i_id)
    one_hop0 = _sc_gather_rows(adj_matrix, pos_i_id, BB // NW)
    i_e = _gather64(emb, one_hop0.reshape(-1)).reshape(BB, KK, DD)
    _, one_hop = _stage1(u_e, pos_e, i_e, one_hop0, gumb1)
    one_hop = one_hop[:, 0]
    pos2_e = _gather64(emb, one_hop)
    one_hop2 = _sc_gather_rows(adj_matrix, one_hop, BB // NW)
    i_e2 = _gather64(emb, one_hop2.reshape(-1)).reshape(BB, KK, DD)
    neg, clog = _stage2a(u_e, pos2_e, i_e2, one_hop2, gumb2, rnd)
    dis_u = _gather64(dis_user_emb, u_id)
    dis_i = _gather64(dis_item_emb, neg.reshape(-1)).reshape(BB, KK, DD)
    gneg, glog = _stage2b(dis_u, dis_i, neg, clog)
    return gneg[:, 0], glog[:, 0]


# speedup vs baseline: 2.3690x; 1.0036x over previous
import functools

import jax
import jax.numpy as jnp
import numpy as np
from jax import lax
from jax.experimental import pallas as pl
from jax.experimental.pallas import tpu as pltpu
from jax.experimental.pallas import tpu_sc as plsc

NN = 50000
NE = 800000
DD = 64
BB = 4096
KK = 32
HI = 24999
BLK = 512


def _sampling_noise():
    key = jax.random.key(42)
    k1, k2, k3 = jax.random.split(key, 3)
    g1 = jax.random.gumbel(k1, (BB, KK), dtype=jnp.float32)
    g2 = jax.random.gumbel(k2, (BB, KK), dtype=jnp.float32)
    rnd = jax.random.randint(k3, (BB, KK), 0, HI + 1, dtype=jnp.int32)
    return g1, g2, rnd


def _stage1_body(u_ref, pos_ref, ie_ref, hop_ref, g_ref, nid_ref, hop_out_ref):
    p = jnp.sum((pos_ref[...][:, None, :] * ie_ref[...])
                * u_ref[...][:, None, :], axis=-1)
    m = jnp.max(p, axis=1, keepdims=True)
    e = jnp.exp(p - m)
    probs = e / jnp.sum(e, axis=1, keepdims=True)
    score = jnp.log(probs + 1e-12) + g_ref[...]
    vmax = jnp.max(score, axis=1, keepdims=True)
    col = jax.lax.broadcasted_iota(jnp.int32, (BLK, KK), 1)
    nid = jnp.min(jnp.where(score == vmax, col, KK), axis=1, keepdims=True)
    nid_ref[...] = nid
    hop_out_ref[...] = jnp.sum(
        jnp.where(col == nid, hop_ref[...], 0), axis=1, keepdims=True)


def _stage1(u_e, pos_e, i_e, one_hop0, gumb1):
    grid = BB // BLK
    return pl.pallas_call(
        _stage1_body,
        grid=(grid,),
        in_specs=[
            pl.BlockSpec((BLK, DD), lambda i: (i, 0)),
            pl.BlockSpec((BLK, DD), lambda i: (i, 0)),
            pl.BlockSpec((BLK, KK, DD), lambda i: (i, 0, 0)),
            pl.BlockSpec((BLK, KK), lambda i: (i, 0)),
            pl.BlockSpec((BLK, KK), lambda i: (i, 0)),
        ],
        out_specs=[
            pl.BlockSpec((BLK, 1), lambda i: (i, 0)),
            pl.BlockSpec((BLK, 1), lambda i: (i, 0)),
        ],
        out_shape=[
            jax.ShapeDtypeStruct((BB, 1), jnp.int32),
            jax.ShapeDtypeStruct((BB, 1), jnp.int32),
        ],
    )(u_e, pos_e, i_e, one_hop0, gumb1)


def _stage2a_body(u_ref, pos_ref, ie_ref, hop_ref, g_ref, rnd_ref,
                  neg_ref, clog_ref):
    p = jnp.sum((pos_ref[...][:, None, :] * ie_ref[...])
                * u_ref[...][:, None, :], axis=-1)
    m = jnp.max(p, axis=1, keepdims=True)
    e = jnp.exp(p - m)
    probs = e / jnp.sum(e, axis=1, keepdims=True)
    score = jnp.log(probs + 1e-12) + g_ref[...]
    col = jax.lax.broadcasted_iota(jnp.int32, (BLK, KK), 1)
    hop = hop_ref[...]
    rnd = rnd_ref[...]
    neg_cols = []
    clog_cols = []
    s = score
    for j in range(KK):
        vmax = jnp.max(s, axis=1, keepdims=True)
        sel = jnp.min(jnp.where(s == vmax, col, KK), axis=1, keepdims=True)
        cand = jnp.sum(jnp.where(col == sel, hop, 0), axis=1, keepdims=True)
        clog = jnp.sum(jnp.where(col == sel, probs, 0.0), axis=1, keepdims=True)
        r = rnd[:, j][:, None]
        cand = jnp.where((cand > HI) | (cand < 0), r, cand)
        neg_cols.append(cand)
        clog_cols.append(clog)
        s = jnp.where(col == sel, -jnp.inf, s)
    neg_ref[...] = jnp.concatenate(neg_cols, axis=1)
    clog_ref[...] = jnp.concatenate(clog_cols, axis=1)


def _stage2a(u_e, pos2_e, i_e2, one_hop2, gumb2, rnd):
    grid = BB // BLK
    return pl.pallas_call(
        _stage2a_body,
        grid=(grid,),
        in_specs=[
            pl.BlockSpec((BLK, DD), lambda i: (i, 0)),
            pl.BlockSpec((BLK, DD), lambda i: (i, 0)),
            pl.BlockSpec((BLK, KK, DD), lambda i: (i, 0, 0)),
            pl.BlockSpec((BLK, KK), lambda i: (i, 0)),
            pl.BlockSpec((BLK, KK), lambda i: (i, 0)),
            pl.BlockSpec((BLK, KK), lambda i: (i, 0)),
        ],
        out_specs=[
            pl.BlockSpec((BLK, KK), lambda i: (i, 0)),
            pl.BlockSpec((BLK, KK), lambda i: (i, 0)),
        ],
        out_shape=[
            jax.ShapeDtypeStruct((BB, KK), jnp.int32),
            jax.ShapeDtypeStruct((BB, KK), jnp.float32),
        ],
    )(u_e, pos2_e, i_e2, one_hop2, gumb2, rnd)


def _stage2b_body(du_ref, di_ref, neg_ref, clog_ref, gneg_ref, glog_ref):
    rank = jnp.sum(di_ref[...] * du_ref[...][:, None, :], axis=-1)
    vmax = jnp.max(rank, axis=1, keepdims=True)
    col = jax.lax.broadcasted_iota(jnp.int32, (BLK, KK), 1)
    idx = jnp.min(jnp.where(rank == vmax, col, KK), axis=1, keepdims=True)
    gneg_ref[...] = jnp.sum(jnp.where(col == idx, neg_ref[...], 0),
                            axis=1, keepdims=True)
    glog_ref[...] = jnp.sum(jnp.where(col == idx, clog_ref[...], 0.0),
                            axis=1, keepdims=True)


def _stage2b(dis_u, dis_i, neg, clog):
    grid = BB // BLK
    return pl.pallas_call(
        _stage2b_body,
        grid=(grid,),
        in_specs=[
            pl.BlockSpec((BLK, DD), lambda i: (i, 0)),
            pl.BlockSpec((BLK, KK, DD), lambda i: (i, 0, 0)),
            pl.BlockSpec((BLK, KK), lambda i: (i, 0)),
            pl.BlockSpec((BLK, KK), lambda i: (i, 0)),
        ],
        out_specs=[
            pl.BlockSpec((BLK, 1), lambda i: (i, 0)),
            pl.BlockSpec((BLK, 1), lambda i: (i, 0)),
        ],
        out_shape=[
            jax.ShapeDtypeStruct((BB, 1), jnp.int32),
            jax.ShapeDtypeStruct((BB, 1), jnp.float32),
        ],
    )(dis_u, dis_i, neg, clog)


NW = 32


@functools.lru_cache(maxsize=None)
def _sc_mesh():
    return plsc.VectorSubcoreMesh(core_axis_name="c", subcore_axis_name="s")


def _sc_gather_rows(table, idx, blk):
    M = idx.shape[0]
    Dw = table.shape[1]
    per = M // NW
    nb = per // blk
    assert per * NW == M and nb * blk == per, (M, blk)

    if nb == 1:
        @functools.partial(
            pl.kernel,
            mesh=_sc_mesh(),
            compiler_params=pltpu.CompilerParams(use_tc_tiling_on_sc=False),
            out_type=jax.ShapeDtypeStruct((M, Dw), table.dtype),
            scratch_types=[
                pltpu.VMEM((blk,), jnp.int32),
                pltpu.VMEM((blk, Dw), table.dtype),
                pltpu.SemaphoreType.DMA,
            ],
        )
        def k1(table_hbm, idx_hbm, out_hbm, idx_v, rows_v, sem):
            wid = lax.axis_index("s") * 2 + lax.axis_index("c")
            base = wid * per
            pltpu.sync_copy(idx_hbm.at[pl.ds(base, blk)], idx_v)
            pltpu.async_copy(table_hbm.at[idx_v], rows_v, sem).wait()
            pltpu.sync_copy(rows_v, out_hbm.at[pl.ds(base, blk)])

        return k1(table, idx)

    assert nb % 2 == 0, nb

    @functools.partial(
        pl.kernel,
        mesh=_sc_mesh(),
        compiler_params=pltpu.CompilerParams(use_tc_tiling_on_sc=False),
        out_type=jax.ShapeDtypeStruct((M, Dw), table.dtype),
        scratch_types=[
            pltpu.VMEM((blk,), jnp.int32),
            pltpu.VMEM((blk,), jnp.int32),
            pltpu.VMEM((blk, Dw), table.dtype),
            pltpu.VMEM((blk, Dw), table.dtype),
            pltpu.SemaphoreType.DMA,
            pltpu.SemaphoreType.DMA,
        ],
    )
    def k(table_hbm, idx_hbm, out_hbm, idx0, idx1, rows0, rows1, s0, s1):
        wid = lax.axis_index("s") * 2 + lax.axis_index("c")
        base = wid * per
        idxs, rows, sems = (idx0, idx1), (rows0, rows1), (s0, s1)
        pltpu.sync_copy(idx_hbm.at[pl.ds(base, blk)], idx0)
        pltpu.async_copy(table_hbm.at[idx0], rows0, s0)

        def outer(g, c):
            for ph in range(2):
                b = g * 2 + ph
                cur_i, cur_r, cur_s = idxs[ph], rows[ph], sems[ph]
                nxt_i, nxt_r, nxt_s = idxs[1 - ph], rows[1 - ph], sems[1 - ph]

                @pl.when(b + 1 < nb)
                def _():
                    pltpu.sync_copy(
                        idx_hbm.at[pl.ds(base + (b + 1) * blk, blk)], nxt_i)
                    pltpu.async_copy(table_hbm.at[nxt_i], nxt_r, nxt_s)

                pltpu.make_async_copy(table_hbm.at[cur_i], cur_r, cur_s).wait()
                pltpu.sync_copy(cur_r, out_hbm.at[pl.ds(base + b * blk, blk)])
            return c

        lax.fori_loop(0, nb // 2, outer, 0)

    return k(table, idx)


def _gather64(table, idx):
    M = idx.shape[0]
    step = NW * 1024 if M >= NW * 1024 else NW * 8
    M_pad = ((M + step - 1) // step) * step
    if M_pad != M:
        idx = jnp.concatenate([idx, jnp.zeros((M_pad - M,), jnp.int32)])
    blk = 512 if M_pad >= NW * 1024 else max(8, M_pad // NW)
    out = _sc_gather_rows(table, idx, blk)
    return out[:M] if M_pad != M else out


NBIN = 50016
MPAD = NW * 26 * 1024
EBLK = 1024


def _dup_stats(d_pad):
    d2 = d_pad.reshape(-1, 16)
    eq = d2[:, :, None] == d2[:, None, :]
    lane = jnp.arange(16, dtype=jnp.int32)
    before = lane[None, :, None] > lane[None, None, :]
    duprank = jnp.sum(eq & before, axis=2, dtype=jnp.int32)
    after = lane[None, :, None] < lane[None, None, :]
    islast = 1 - jnp.max((eq & after).astype(jnp.int32), axis=2)
    return duprank.reshape(-1), islast.reshape(-1)


def _sc_hist(d_pad, duprank, islast):
    per = MPAD // NW
    nb = per // EBLK

    @functools.partial(
        pl.kernel,
        mesh=_sc_mesh(),
        compiler_params=pltpu.CompilerParams(use_tc_tiling_on_sc=False, needs_layout_passes=False),
        out_type=jax.ShapeDtypeStruct((NW, NBIN), jnp.int32),
        scratch_types=[
            pltpu.VMEM((EBLK,), jnp.int32),
            pltpu.VMEM((EBLK,), jnp.int32),
            pltpu.VMEM((EBLK,), jnp.int32),
            pltpu.VMEM((NBIN,), jnp.int32),
        ],
    )
    def k(d_hbm, dr_hbm, il_hbm, out_hbm, d_v, dr_v, il_v, hist_v):
        wid = lax.axis_index("s") * 2 + lax.axis_index("c")
        base = wid * per

        def zero_body(i, c):
            hist_v[pl.ds(i * 16, 16)] = jnp.zeros((16,), jnp.int32)
            return c

        lax.fori_loop(0, NBIN // 16, zero_body, 0)

        def blk_body(bI, c):
            off = base + bI * EBLK
            pltpu.sync_copy(d_hbm.at[pl.ds(off, EBLK)], d_v)
            pltpu.sync_copy(dr_hbm.at[pl.ds(off, EBLK)], dr_v)
            pltpu.sync_copy(il_hbm.at[pl.ds(off, EBLK)], il_v)

            def e_body(j, c2):
                d16 = d_v[pl.ds(j * 16, 16)]
                cnt = plsc.load_gather(hist_v, [d16])
                cnt = cnt + dr_v[pl.ds(j * 16, 16)] + 1
                mask = il_v[pl.ds(j * 16, 16)] != 0
                plsc.store_scatter(hist_v, [d16], cnt, mask=mask)
                return c2

            lax.fori_loop(0, EBLK // 16, e_body, 0)
            return c

        lax.fori_loop(0, nb, blk_body, 0)
        pltpu.sync_copy(hist_v, out_hbm.at[wid])

    return k(d_pad, duprank, islast)


def _sc_rank_permute(d_pad, s_pad, duprank, islast, dinv_p, basew):
    per = MPAD // NW
    nb = per // EBLK

    @functools.partial(
        pl.kernel,
        mesh=_sc_mesh(),
        compiler_params=pltpu.CompilerParams(use_tc_tiling_on_sc=False, needs_layout_passes=False),
        out_type=[
            jax.ShapeDtypeStruct((MPAD,), jnp.int32),
            jax.ShapeDtypeStruct((MPAD,), jnp.int32),
            jax.ShapeDtypeStruct((MPAD,), jnp.float32),
        ],
        scratch_types=[
            pltpu.VMEM((EBLK,), jnp.int32),
            pltpu.VMEM((EBLK,), jnp.int32),
            pltpu.VMEM((EBLK,), jnp.int32),
            pltpu.VMEM((EBLK,), jnp.int32),
            pltpu.VMEM((EBLK,), jnp.int32),
            pltpu.VMEM((EBLK,), jnp.float32),
            pltpu.VMEM((NBIN,), jnp.int32),
            pltpu.VMEM((NBIN,), jnp.float32),
            pltpu.SemaphoreType.DMA,
            pltpu.SemaphoreType.DMA,
            pltpu.SemaphoreType.DMA,
        ],
    )
    def k(d_hbm, s_hbm, dr_hbm, il_hbm, dinv_hbm, basew_hbm,
          ss_hbm, ds_hbm, ns_hbm,
          d_v, s_v, dr_v, il_v, pos_v, norm_v, base_v, dinv_v,
          sem1, sem2, sem3):
        wid = lax.axis_index("s") * 2 + lax.axis_index("c")
        base = wid * per
        pltpu.sync_copy(basew_hbm.at[wid], base_v)
        pltpu.sync_copy(dinv_hbm, dinv_v)

        def blk_body(bI, c):
            off = base + bI * EBLK
            pltpu.sync_copy(d_hbm.at[pl.ds(off, EBLK)], d_v)
            pltpu.sync_copy(s_hbm.at[pl.ds(off, EBLK)], s_v)
            pltpu.sync_copy(dr_hbm.at[pl.ds(off, EBLK)], dr_v)
            pltpu.sync_copy(il_hbm.at[pl.ds(off, EBLK)], il_v)

            def e_body(j, c2):
                sl = pl.ds(j * 16, 16)
                d16 = d_v[sl]
                s16 = s_v[sl]
                p16 = plsc.load_gather(base_v, [d16]) + dr_v[sl]
                mask = il_v[sl] != 0
                plsc.store_scatter(base_v, [d16], p16 + 1, mask=mask)
                pos_v[sl] = p16
                ds_ = plsc.load_gather(dinv_v, [s16])
                dd_ = plsc.load_gather(dinv_v, [d16])
                norm_v[sl] = ds_ * dd_
                return c2

            lax.fori_loop(0, EBLK // 16, e_body, 0)
            h1 = pltpu.async_copy(s_v, ss_hbm.at[pos_v], sem1)
            h2 = pltpu.async_copy(d_v, ds_hbm.at[pos_v], sem2)
            h3 = pltpu.async_copy(norm_v, ns_hbm.at[pos_v], sem3)
            h1.wait()
            h2.wait()
            h3.wait()
            return c

        lax.fori_loop(0, nb, blk_body, 0)

    return k(d_pad, s_pad, duprank, islast, dinv_p, basew)


def _sorted_edges(src, dst):
    loop = jnp.arange(NN)
    npad = MPAD - (NE + NN)
    d_pad = jnp.concatenate([dst, loop, jnp.full((npad,), NBIN - 2, jnp.int32)])
    s_pad = jnp.concatenate([src, loop, jnp.zeros((npad,), jnp.int32)])
    duprank, islast = _dup_stats(d_pad)
    hist = _sc_hist(d_pad, duprank, islast)
    tot = jnp.sum(hist, axis=0)
    deg = tot[:NN].astype(jnp.float32)
    dinv = jnp.where(deg > 0, deg ** -0.5, 0.0)
    dinv_p = jnp.concatenate([dinv, jnp.zeros((NBIN - NN,), jnp.float32)])
    off = jnp.cumsum(tot) - tot
    basew = off[None, :] + (jnp.cumsum(hist, axis=0) - hist)
    ss, ds, ns = _sc_rank_permute(d_pad, s_pad, duprank, islast, dinv_p, basew)
    return ss, ds[: NE + NN], ns[: NE + NN]


def _conv(x, W, b, s_sorted, d_sorted, norm_sorted):
    xw = x @ W
    msg = _gather64(xw, s_sorted)[: NE + NN] * norm_sorted[:, None]
    out = lax.scatter_add(
        jnp.zeros((NN, DD), jnp.float32),
        d_sorted[:, None],
        msg,
        lax.ScatterDimensionNumbers(
            update_window_dims=(1,),
            inserted_window_dims=(0,),
            scatter_dims_to_operand_dims=(0,),
        ),
        indices_are_sorted=True,
        unique_indices=False,
    )
    return out + b


def _bn(x, g, beta, eps=1e-5):
    m = jnp.mean(x, axis=0)
    v = jnp.var(x, axis=0)
    return (x - m) / jnp.sqrt(v + eps) * g + beta


def _gcn(x, edges, W1, b1, g1, beta1, W2, b2, g2, beta2):
    src, dst = edges[0], edges[1]
    ss, ds, ns = _sorted_edges(src, dst)
    h = _conv(x, W1, b1, ss, ds, ns)
    h = jax.nn.leaky_relu(h, 0.01)
    h = _bn(h, g1, beta1)
    h = _conv(h, W2, b2, ss, ds, ns)
    h = _bn(h, g2, beta2)
    return h


def kernel(u_id, pos_i_id, adj_matrix, edges, entity_embedding,
           W1, b1, g1, beta1, W2, b2, g2, beta2,
           dis_user_emb, dis_item_emb):
    gumb1, gumb2, rnd = _sampling_noise()
    emb = _gcn(entity_embedding, edges, W1, b1, g1, beta1, W2, b2, g2, beta2)
    u_e = _gather64(emb, u_id)
    pos_e = _gather64(emb, pos_i_id)
    one_hop0 = _sc_gather_rows(adj_matrix, pos_i_id, BB // NW)
    i_e = _gather64(emb, one_hop0.reshape(-1)).reshape(BB, KK, DD)
    _, one_hop = _stage1(u_e, pos_e, i_e, one_hop0, gumb1)
    one_hop = one_hop[:, 0]
    pos2_e = _gather64(emb, one_hop)
    one_hop2 = _sc_gather_rows(adj_matrix, one_hop, BB // NW)
    i_e2 = _gather64(emb, one_hop2.reshape(-1)).reshape(BB, KK, DD)
    neg, clog = _stage2a(u_e, pos2_e, i_e2, one_hop2, gumb2, rnd)
    dis_u = _gather64(dis_user_emb, u_id)
    dis_i = _gather64(dis_item_emb, neg.reshape(-1)).reshape(BB, KK, DD)
    gneg, glog = _stage2b(dis_u, dis_i, neg, clog)
    return gneg[:, 0], glog[:, 0]


# speedup vs baseline: 2.4259x; 1.0240x over previous
import functools

import jax
import jax.numpy as jnp
import numpy as np
from jax import lax
from jax.experimental import pallas as pl
from jax.experimental.pallas import tpu as pltpu
from jax.experimental.pallas import tpu_sc as plsc

NN = 50000
NE = 800000
DD = 64
BB = 4096
KK = 32
HI = 24999
BLK = 512


def _sampling_noise():
    key = jax.random.key(42)
    k1, k2, k3 = jax.random.split(key, 3)
    g1 = jax.random.gumbel(k1, (BB, KK), dtype=jnp.float32)
    g2 = jax.random.gumbel(k2, (BB, KK), dtype=jnp.float32)
    rnd = jax.random.randint(k3, (BB, KK), 0, HI + 1, dtype=jnp.int32)
    return g1, g2, rnd


def _stage1_body(u_ref, pos_ref, ie_ref, hop_ref, g_ref, nid_ref, hop_out_ref):
    p = jnp.sum((pos_ref[...][:, None, :] * ie_ref[...])
                * u_ref[...][:, None, :], axis=-1)
    m = jnp.max(p, axis=1, keepdims=True)
    e = jnp.exp(p - m)
    probs = e / jnp.sum(e, axis=1, keepdims=True)
    score = jnp.log(probs + 1e-12) + g_ref[...]
    vmax = jnp.max(score, axis=1, keepdims=True)
    col = jax.lax.broadcasted_iota(jnp.int32, (BLK, KK), 1)
    nid = jnp.min(jnp.where(score == vmax, col, KK), axis=1, keepdims=True)
    nid_ref[...] = nid
    hop_out_ref[...] = jnp.sum(
        jnp.where(col == nid, hop_ref[...], 0), axis=1, keepdims=True)


def _stage1(u_e, pos_e, i_e, one_hop0, gumb1):
    grid = BB // BLK
    return pl.pallas_call(
        _stage1_body,
        grid=(grid,),
        in_specs=[
            pl.BlockSpec((BLK, DD), lambda i: (i, 0)),
            pl.BlockSpec((BLK, DD), lambda i: (i, 0)),
            pl.BlockSpec((BLK, KK, DD), lambda i: (i, 0, 0)),
            pl.BlockSpec((BLK, KK), lambda i: (i, 0)),
            pl.BlockSpec((BLK, KK), lambda i: (i, 0)),
        ],
        out_specs=[
            pl.BlockSpec((BLK, 1), lambda i: (i, 0)),
            pl.BlockSpec((BLK, 1), lambda i: (i, 0)),
        ],
        out_shape=[
            jax.ShapeDtypeStruct((BB, 1), jnp.int32),
            jax.ShapeDtypeStruct((BB, 1), jnp.int32),
        ],
    )(u_e, pos_e, i_e, one_hop0, gumb1)


def _stage2a_body(u_ref, pos_ref, ie_ref, hop_ref, g_ref, rnd_ref,
                  neg_ref, clog_ref):
    p = jnp.sum((pos_ref[...][:, None, :] * ie_ref[...])
                * u_ref[...][:, None, :], axis=-1)
    m = jnp.max(p, axis=1, keepdims=True)
    e = jnp.exp(p - m)
    probs = e / jnp.sum(e, axis=1, keepdims=True)
    score = jnp.log(probs + 1e-12) + g_ref[...]
    col = jax.lax.broadcasted_iota(jnp.int32, (BLK, KK), 1)
    hop = hop_ref[...]
    rnd = rnd_ref[...]
    neg_cols = []
    clog_cols = []
    s = score
    for j in range(KK):
        vmax = jnp.max(s, axis=1, keepdims=True)
        sel = jnp.min(jnp.where(s == vmax, col, KK), axis=1, keepdims=True)
        cand = jnp.sum(jnp.where(col == sel, hop, 0), axis=1, keepdims=True)
        clog = jnp.sum(jnp.where(col == sel, probs, 0.0), axis=1, keepdims=True)
        r = rnd[:, j][:, None]
        cand = jnp.where((cand > HI) | (cand < 0), r, cand)
        neg_cols.append(cand)
        clog_cols.append(clog)
        s = jnp.where(col == sel, -jnp.inf, s)
    neg_ref[...] = jnp.concatenate(neg_cols, axis=1)
    clog_ref[...] = jnp.concatenate(clog_cols, axis=1)


def _stage2a(u_e, pos2_e, i_e2, one_hop2, gumb2, rnd):
    grid = BB // BLK
    return pl.pallas_call(
        _stage2a_body,
        grid=(grid,),
        in_specs=[
            pl.BlockSpec((BLK, DD), lambda i: (i, 0)),
            pl.BlockSpec((BLK, DD), lambda i: (i, 0)),
            pl.BlockSpec((BLK, KK, DD), lambda i: (i, 0, 0)),
            pl.BlockSpec((BLK, KK), lambda i: (i, 0)),
            pl.BlockSpec((BLK, KK), lambda i: (i, 0)),
            pl.BlockSpec((BLK, KK), lambda i: (i, 0)),
        ],
        out_specs=[
            pl.BlockSpec((BLK, KK), lambda i: (i, 0)),
            pl.BlockSpec((BLK, KK), lambda i: (i, 0)),
        ],
        out_shape=[
            jax.ShapeDtypeStruct((BB, KK), jnp.int32),
            jax.ShapeDtypeStruct((BB, KK), jnp.float32),
        ],
    )(u_e, pos2_e, i_e2, one_hop2, gumb2, rnd)


def _stage2b_body(du_ref, di_ref, neg_ref, clog_ref, gneg_ref, glog_ref):
    rank = jnp.sum(di_ref[...] * du_ref[...][:, None, :], axis=-1)
    vmax = jnp.max(rank, axis=1, keepdims=True)
    col = jax.lax.broadcasted_iota(jnp.int32, (BLK, KK), 1)
    idx = jnp.min(jnp.where(rank == vmax, col, KK), axis=1, keepdims=True)
    gneg_ref[...] = jnp.sum(jnp.where(col == idx, neg_ref[...], 0),
                            axis=1, keepdims=True)
    glog_ref[...] = jnp.sum(jnp.where(col == idx, clog_ref[...], 0.0),
                            axis=1, keepdims=True)


def _stage2b(dis_u, dis_i, neg, clog):
    grid = BB // BLK
    return pl.pallas_call(
        _stage2b_body,
        grid=(grid,),
        in_specs=[
            pl.BlockSpec((BLK, DD), lambda i: (i, 0)),
            pl.BlockSpec((BLK, KK, DD), lambda i: (i, 0, 0)),
            pl.BlockSpec((BLK, KK), lambda i: (i, 0)),
            pl.BlockSpec((BLK, KK), lambda i: (i, 0)),
        ],
        out_specs=[
            pl.BlockSpec((BLK, 1), lambda i: (i, 0)),
            pl.BlockSpec((BLK, 1), lambda i: (i, 0)),
        ],
        out_shape=[
            jax.ShapeDtypeStruct((BB, 1), jnp.int32),
            jax.ShapeDtypeStruct((BB, 1), jnp.float32),
        ],
    )(dis_u, dis_i, neg, clog)


NW = 32


@functools.lru_cache(maxsize=None)
def _sc_mesh():
    return plsc.VectorSubcoreMesh(core_axis_name="c", subcore_axis_name="s")


def _sc_gather_rows(table, idx, blk):
    M = idx.shape[0]
    Dw = table.shape[1]
    per = M // NW
    nb = per // blk
    assert per * NW == M and nb * blk == per, (M, blk)
    cp = pltpu.CompilerParams(use_tc_tiling_on_sc=(Dw == 128))

    if nb == 1:
        @functools.partial(
            pl.kernel,
            mesh=_sc_mesh(),
            compiler_params=cp,
            out_type=jax.ShapeDtypeStruct((M, Dw), table.dtype),
            scratch_types=[
                pltpu.VMEM((blk,), jnp.int32),
                pltpu.VMEM((blk, Dw), table.dtype),
                pltpu.SemaphoreType.DMA,
            ],
        )
        def k1(table_hbm, idx_hbm, out_hbm, idx_v, rows_v, sem):
            wid = lax.axis_index("s") * 2 + lax.axis_index("c")
            base = wid * per
            pltpu.sync_copy(idx_hbm.at[pl.ds(base, blk)], idx_v)
            pltpu.async_copy(table_hbm.at[idx_v], rows_v, sem).wait()
            pltpu.sync_copy(rows_v, out_hbm.at[pl.ds(base, blk)])

        return k1(table, idx)

    assert nb % 2 == 0, nb

    @functools.partial(
        pl.kernel,
        mesh=_sc_mesh(),
        compiler_params=cp,
        out_type=jax.ShapeDtypeStruct((M, Dw), table.dtype),
        scratch_types=[
            pltpu.VMEM((blk,), jnp.int32),
            pltpu.VMEM((blk,), jnp.int32),
            pltpu.VMEM((blk, Dw), table.dtype),
            pltpu.VMEM((blk, Dw), table.dtype),
            pltpu.SemaphoreType.DMA,
            pltpu.SemaphoreType.DMA,
        ],
    )
    def k(table_hbm, idx_hbm, out_hbm, idx0, idx1, rows0, rows1, s0, s1):
        wid = lax.axis_index("s") * 2 + lax.axis_index("c")
        base = wid * per
        idxs, rows, sems = (idx0, idx1), (rows0, rows1), (s0, s1)
        pltpu.sync_copy(idx_hbm.at[pl.ds(base, blk)], idx0)
        pltpu.async_copy(table_hbm.at[idx0], rows0, s0)

        def outer(g, c):
            for ph in range(2):
                b = g * 2 + ph
                cur_i, cur_r, cur_s = idxs[ph], rows[ph], sems[ph]
                nxt_i, nxt_r, nxt_s = idxs[1 - ph], rows[1 - ph], sems[1 - ph]

                @pl.when(b + 1 < nb)
                def _():
                    pltpu.sync_copy(
                        idx_hbm.at[pl.ds(base + (b + 1) * blk, blk)], nxt_i)
                    pltpu.async_copy(table_hbm.at[nxt_i], nxt_r, nxt_s)

                pltpu.make_async_copy(table_hbm.at[cur_i], cur_r, cur_s).wait()
                pltpu.sync_copy(cur_r, out_hbm.at[pl.ds(base + b * blk, blk)])
            return c

        lax.fori_loop(0, nb // 2, outer, 0)

    return k(table, idx)


def _pad128(table):
    return jnp.pad(table, ((0, 0), (0, 128 - table.shape[1])))


def _gather64(table128, idx):
    M = idx.shape[0]
    step = NW * 1024 if M >= NW * 1024 else NW * 8
    M_pad = ((M + step - 1) // step) * step
    if M_pad != M:
        idx = jnp.concatenate([idx, jnp.zeros((M_pad - M,), jnp.int32)])
    blk = 256 if M_pad >= NW * 1024 else max(8, M_pad // NW)
    out = _sc_gather_rows(table128, idx, blk)
    return out[:M, :DD]


NBIN = 50016
MPAD = NW * 26 * 1024
EBLK = 1024


def _dup_stats(d_pad):
    d2 = d_pad.reshape(-1, 16)
    eq = d2[:, :, None] == d2[:, None, :]
    lane = jnp.arange(16, dtype=jnp.int32)
    before = lane[None, :, None] > lane[None, None, :]
    duprank = jnp.sum(eq & before, axis=2, dtype=jnp.int32)
    after = lane[None, :, None] < lane[None, None, :]
    islast = 1 - jnp.max((eq & after).astype(jnp.int32), axis=2)
    return duprank.reshape(-1), islast.reshape(-1)


def _sc_hist(d_pad, duprank, islast):
    per = MPAD // NW
    nb = per // EBLK

    @functools.partial(
        pl.kernel,
        mesh=_sc_mesh(),
        compiler_params=pltpu.CompilerParams(use_tc_tiling_on_sc=False, needs_layout_passes=False),
        out_type=jax.ShapeDtypeStruct((NW, NBIN), jnp.int32),
        scratch_types=[
            pltpu.VMEM((EBLK,), jnp.int32),
            pltpu.VMEM((EBLK,), jnp.int32),
            pltpu.VMEM((EBLK,), jnp.int32),
            pltpu.VMEM((NBIN,), jnp.int32),
        ],
    )
    def k(d_hbm, dr_hbm, il_hbm, out_hbm, d_v, dr_v, il_v, hist_v):
        wid = lax.axis_index("s") * 2 + lax.axis_index("c")
        base = wid * per

        def zero_body(i, c):
            hist_v[pl.ds(i * 16, 16)] = jnp.zeros((16,), jnp.int32)
            return c

        lax.fori_loop(0, NBIN // 16, zero_body, 0)

        def blk_body(bI, c):
            off = base + bI * EBLK
            pltpu.sync_copy(d_hbm.at[pl.ds(off, EBLK)], d_v)
            pltpu.sync_copy(dr_hbm.at[pl.ds(off, EBLK)], dr_v)
            pltpu.sync_copy(il_hbm.at[pl.ds(off, EBLK)], il_v)

            def e_body(j, c2):
                d16 = d_v[pl.ds(j * 16, 16)]
                cnt = plsc.load_gather(hist_v, [d16])
                cnt = cnt + dr_v[pl.ds(j * 16, 16)] + 1
                mask = il_v[pl.ds(j * 16, 16)] != 0
                plsc.store_scatter(hist_v, [d16], cnt, mask=mask)
                return c2

            lax.fori_loop(0, EBLK // 16, e_body, 0)
            return c

        lax.fori_loop(0, nb, blk_body, 0)
        pltpu.sync_copy(hist_v, out_hbm.at[wid])

    return k(d_pad, duprank, islast)


def _sc_rank_permute(d_pad, s_pad, duprank, islast, dinv_p, basew):
    per = MPAD // NW
    nb = per // EBLK

    @functools.partial(
        pl.kernel,
        mesh=_sc_mesh(),
        compiler_params=pltpu.CompilerParams(use_tc_tiling_on_sc=False, needs_layout_passes=False),
        out_type=[
            jax.ShapeDtypeStruct((MPAD,), jnp.int32),
            jax.ShapeDtypeStruct((MPAD,), jnp.int32),
            jax.ShapeDtypeStruct((MPAD,), jnp.float32),
        ],
        scratch_types=[
            pltpu.VMEM((EBLK,), jnp.int32),
            pltpu.VMEM((EBLK,), jnp.int32),
            pltpu.VMEM((EBLK,), jnp.int32),
            pltpu.VMEM((EBLK,), jnp.int32),
            pltpu.VMEM((EBLK,), jnp.int32),
            pltpu.VMEM((EBLK,), jnp.float32),
            pltpu.VMEM((NBIN,), jnp.int32),
            pltpu.VMEM((NBIN,), jnp.float32),
            pltpu.SemaphoreType.DMA,
            pltpu.SemaphoreType.DMA,
            pltpu.SemaphoreType.DMA,
        ],
    )
    def k(d_hbm, s_hbm, dr_hbm, il_hbm, dinv_hbm, basew_hbm,
          ss_hbm, ds_hbm, ns_hbm,
          d_v, s_v, dr_v, il_v, pos_v, norm_v, base_v, dinv_v,
          sem1, sem2, sem3):
        wid = lax.axis_index("s") * 2 + lax.axis_index("c")
        base = wid * per
        pltpu.sync_copy(basew_hbm.at[wid], base_v)
        pltpu.sync_copy(dinv_hbm, dinv_v)

        def blk_body(bI, c):
            off = base + bI * EBLK
            pltpu.sync_copy(d_hbm.at[pl.ds(off, EBLK)], d_v)
            pltpu.sync_copy(s_hbm.at[pl.ds(off, EBLK)], s_v)
            pltpu.sync_copy(dr_hbm.at[pl.ds(off, EBLK)], dr_v)
            pltpu.sync_copy(il_hbm.at[pl.ds(off, EBLK)], il_v)

            def e_body(j, c2):
                sl = pl.ds(j * 16, 16)
                d16 = d_v[sl]
                s16 = s_v[sl]
                p16 = plsc.load_gather(base_v, [d16]) + dr_v[sl]
                mask = il_v[sl] != 0
                plsc.store_scatter(base_v, [d16], p16 + 1, mask=mask)
                pos_v[sl] = p16
                ds_ = plsc.load_gather(dinv_v, [s16])
                dd_ = plsc.load_gather(dinv_v, [d16])
                norm_v[sl] = ds_ * dd_
                return c2

            lax.fori_loop(0, EBLK // 16, e_body, 0)
            h1 = pltpu.async_copy(s_v, ss_hbm.at[pos_v], sem1)
            h2 = pltpu.async_copy(d_v, ds_hbm.at[pos_v], sem2)
            h3 = pltpu.async_copy(norm_v, ns_hbm.at[pos_v], sem3)
            h1.wait()
            h2.wait()
            h3.wait()
            return c

        lax.fori_loop(0, nb, blk_body, 0)

    return k(d_pad, s_pad, duprank, islast, dinv_p, basew)


def _sorted_edges(src, dst):
    loop = jnp.arange(NN)
    npad = MPAD - (NE + NN)
    d_pad = jnp.concatenate([dst, loop, jnp.full((npad,), NBIN - 2, jnp.int32)])
    s_pad = jnp.concatenate([src, loop, jnp.zeros((npad,), jnp.int32)])
    duprank, islast = _dup_stats(d_pad)
    hist = _sc_hist(d_pad, duprank, islast)
    tot = jnp.sum(hist, axis=0)
    deg = tot[:NN].astype(jnp.float32)
    dinv = jnp.where(deg > 0, deg ** -0.5, 0.0)
    dinv_p = jnp.concatenate([dinv, jnp.zeros((NBIN - NN,), jnp.float32)])
    off = jnp.cumsum(tot) - tot
    basew = off[None, :] + (jnp.cumsum(hist, axis=0) - hist)
    ss, ds, ns = _sc_rank_permute(d_pad, s_pad, duprank, islast, dinv_p, basew)
    return ss, ds[: NE + NN], ns[: NE + NN]


def _conv(x, W, b, s_sorted, d_sorted, norm_sorted):
    xw = x @ W
    msg = _gather64(_pad128(xw), s_sorted)[: NE + NN] * norm_sorted[:, None]
    out = lax.scatter_add(
        jnp.zeros((NN, DD), jnp.float32),
        d_sorted[:, None],
        msg,
        lax.ScatterDimensionNumbers(
            update_window_dims=(1,),
            inserted_window_dims=(0,),
            scatter_dims_to_operand_dims=(0,),
        ),
        indices_are_sorted=True,
        unique_indices=False,
    )
    return out + b


def _bn(x, g, beta, eps=1e-5):
    m = jnp.mean(x, axis=0)
    v = jnp.var(x, axis=0)
    return (x - m) / jnp.sqrt(v + eps) * g + beta


def _gcn(x, edges, W1, b1, g1, beta1, W2, b2, g2, beta2):
    src, dst = edges[0], edges[1]
    ss, ds, ns = _sorted_edges(src, dst)
    h = _conv(x, W1, b1, ss, ds, ns)
    h = jax.nn.leaky_relu(h, 0.01)
    h = _bn(h, g1, beta1)
    h = _conv(h, W2, b2, ss, ds, ns)
    h = _bn(h, g2, beta2)
    return h


def kernel(u_id, pos_i_id, adj_matrix, edges, entity_embedding,
           W1, b1, g1, beta1, W2, b2, g2, beta2,
           dis_user_emb, dis_item_emb):
    gumb1, gumb2, rnd = _sampling_noise()
    emb = _gcn(entity_embedding, edges, W1, b1, g1, beta1, W2, b2, g2, beta2)
    emb128 = _pad128(emb)
    u_e = _gather64(emb128, u_id)
    pos_e = _gather64(emb128, pos_i_id)
    one_hop0 = _sc_gather_rows(adj_matrix, pos_i_id, BB // NW)
    i_e = _gather64(emb128, one_hop0.reshape(-1)).reshape(BB, KK, DD)
    _, one_hop = _stage1(u_e, pos_e, i_e, one_hop0, gumb1)
    one_hop = one_hop[:, 0]
    pos2_e = _gather64(emb128, one_hop)
    one_hop2 = _sc_gather_rows(adj_matrix, one_hop, BB // NW)
    i_e2 = _gather64(emb128, one_hop2.reshape(-1)).reshape(BB, KK, DD)
    neg, clog = _stage2a(u_e, pos2_e, i_e2, one_hop2, gumb2, rnd)
    dis_u = _gather64(_pad128(dis_user_emb), u_id)
    dis_i = _gather64(_pad128(dis_item_emb), neg.reshape(-1)).reshape(BB, KK, DD)
    gneg, glog = _stage2b(dis_u, dis_i, neg, clog)
    return gneg[:, 0], glog[:, 0]
